# baseline (speedup 1.0000x reference)
import functools

import jax
import jax.numpy as jnp
from jax import lax
from jax.experimental import pallas as pl

_SCALE = 0.5
_ROWS_PER_BLOCK = 1024


def _body(depth, in_ref, anc_ref, out_ref):
    x = in_ref[...]
    r, w = x.shape
    lx = x[:, :depth]
    lcols = lax.broadcasted_iota(jnp.int32, (r, depth), 1)
    m = jnp.max(lx, axis=1, keepdims=True)
    idx = jnp.min(jnp.where(lx == m, lcols, depth), axis=1, keepdims=True)
    onehot = lcols == idx
    anc = jnp.sum(jnp.where(onehot, anc_ref[...], 0.0), axis=1, keepdims=True)
    cols = lax.broadcasted_iota(jnp.int32, (r, w), 1)
    maskv = cols == idx + depth
    sv = jnp.sum(jnp.where(maskv, x, 0.0), axis=1, keepdims=True)
    out_ref[...] = anc + sv * _SCALE


def kernel(inputs, degAnchor):
    b, w = inputs.shape
    depth = degAnchor.shape[0]
    r = _ROWS_PER_BLOCK
    out = pl.pallas_call(
        functools.partial(_body, depth),
        grid=(b // r,),
        in_specs=[
            pl.BlockSpec((r, w), lambda i: (i, 0)),
            pl.BlockSpec((1, depth), lambda i: (0, 0)),
        ],
        out_specs=pl.BlockSpec((r, 1), lambda i: (i, 0)),
        out_shape=jax.ShapeDtypeStruct((b, 1), jnp.float32),
    )(inputs, degAnchor[None, :])
    return out[:, 0]


# speedup vs baseline: 1.0434x; 1.0434x over previous
import functools

import jax
import jax.numpy as jnp
from jax import lax
from jax.experimental import pallas as pl

_SCALE = 0.5
_ROWS_PER_BLOCK = 1024


def _body(depth, in_ref, anc_ref, out_ref):
    x = in_ref[...]
    r, w = x.shape
    lx = x[:, :depth]
    lcols = lax.broadcasted_iota(jnp.int32, (r, depth), 1)
    m = jnp.max(lx, axis=1, keepdims=True)
    idx = jnp.min(jnp.where(lx == m, lcols, depth), axis=1, keepdims=True)
    onehot = lcols == idx
    anc = jnp.sum(jnp.where(onehot, anc_ref[...], 0.0), axis=1, keepdims=True)
    cols = lax.broadcasted_iota(jnp.int32, (r, w), 1)
    maskv = cols == idx + depth
    sv = jnp.sum(jnp.where(maskv, x, 0.0), axis=1, keepdims=True)
    out_ref[...] = (anc + sv * _SCALE)[:, 0]


def kernel(inputs, degAnchor):
    b, w = inputs.shape
    depth = degAnchor.shape[0]
    r = _ROWS_PER_BLOCK
    out = pl.pallas_call(
        functools.partial(_body, depth),
        grid=(b // r,),
        in_specs=[
            pl.BlockSpec((r, w), lambda i: (i, 0)),
            pl.BlockSpec((1, depth), lambda i: (0, 0)),
        ],
        out_specs=pl.BlockSpec((r,), lambda i: (i,)),
        out_shape=jax.ShapeDtypeStruct((b,), jnp.float32),
    )(inputs, degAnchor[None, :])
    return out


# speedup vs baseline: 1.1097x; 1.0635x over previous
import functools

import jax
import jax.numpy as jnp
from jax import lax
from jax.experimental import pallas as pl

_SCALE = 0.5
_ROWS_PER_BLOCK = 2048


def _body(depth, in_ref, anc_ref, out_ref):
    x = in_ref[...]
    r, w = x.shape
    lx = x[:, :depth]
    lcols = lax.broadcasted_iota(jnp.int32, (r, depth), 1)
    m = jnp.max(lx, axis=1, keepdims=True)
    idx = jnp.min(jnp.where(lx == m, lcols, depth), axis=1, keepdims=True)
    onehot = lcols == idx
    anc = jnp.sum(jnp.where(onehot, anc_ref[...], 0.0), axis=1, keepdims=True)
    cols = lax.broadcasted_iota(jnp.int32, (r, w), 1)
    maskv = cols == idx + depth
    sv = jnp.sum(jnp.where(maskv, x, 0.0), axis=1, keepdims=True)
    out_ref[...] = (anc + sv * _SCALE)[:, 0]


def kernel(inputs, degAnchor):
    b, w = inputs.shape
    depth = degAnchor.shape[0]
    r = _ROWS_PER_BLOCK
    out = pl.pallas_call(
        functools.partial(_body, depth),
        grid=(b // r,),
        in_specs=[
            pl.BlockSpec((r, w), lambda i: (i, 0)),
            pl.BlockSpec((1, depth), lambda i: (0, 0)),
        ],
        out_specs=pl.BlockSpec((r,), lambda i: (i,)),
        out_shape=jax.ShapeDtypeStruct((b,), jnp.float32),
    )(inputs, degAnchor[None, :])
    return out


# speedup vs baseline: 1.1121x; 1.0021x over previous
import functools

import jax
import jax.numpy as jnp
from jax import lax
from jax.experimental import pallas as pl

_SCALE = 0.5
_ROWS_PER_BLOCK = 4096


def _body(depth, in_ref, anc_ref, out_ref):
    x = in_ref[...]
    r, w = x.shape
    lx = x[:, :depth]
    lcols = lax.broadcasted_iota(jnp.int32, (r, depth), 1)
    m = jnp.max(lx, axis=1, keepdims=True)
    idx = jnp.min(jnp.where(lx == m, lcols, depth), axis=1, keepdims=True)
    onehot = lcols == idx
    anc = jnp.sum(jnp.where(onehot, anc_ref[...], 0.0), axis=1, keepdims=True)
    cols = lax.broadcasted_iota(jnp.int32, (r, w), 1)
    maskv = cols == idx + depth
    sv = jnp.sum(jnp.where(maskv, x, 0.0), axis=1, keepdims=True)
    out_ref[...] = (anc + sv * _SCALE)[:, 0]


def kernel(inputs, degAnchor):
    b, w = inputs.shape
    depth = degAnchor.shape[0]
    r = _ROWS_PER_BLOCK
    out = pl.pallas_call(
        functools.partial(_body, depth),
        grid=(b // r,),
        in_specs=[
            pl.BlockSpec((r, w), lambda i: (i, 0)),
            pl.BlockSpec((1, depth), lambda i: (0, 0)),
        ],
        out_specs=pl.BlockSpec((r,), lambda i: (i,)),
        out_shape=jax.ShapeDtypeStruct((b,), jnp.float32),
    )(inputs, degAnchor[None, :])
    return out
